# baseline (speedup 1.0000x reference)
import functools

import jax
import jax.numpy as jnp
from jax import lax
from jax.experimental import pallas as pl
from jax.experimental.pallas import tpu as pltpu
from jax.experimental.pallas import tpu_sc as plsc

VOCAB = 100000
EDIM = 32
NTAB = 26
B = 16384
NUM_DENSE = 13
CAT_DIM = NTAB * EDIM

BC = 4096
NCHUNK = B // BC
BN = 4096


def _sc_gather_t(tabT, idx):
    mesh = plsc.VectorSubcoreMesh(core_axis_name="core", subcore_axis_name="subcore")

    @functools.partial(
        pl.kernel,
        out_type=jax.ShapeDtypeStruct((CAT_DIM, B), jnp.float32),
        mesh=mesh,
        compiler_params=pltpu.CompilerParams(needs_layout_passes=False),
        scratch_types=[
            pltpu.VMEM((VOCAB,), jnp.float32),
            pltpu.VMEM((B,), jnp.int32),
            pltpu.VMEM((BC,), jnp.float32),
            pltpu.VMEM((BC,), jnp.float32),
            pltpu.SemaphoreType.DMA,
            pltpu.SemaphoreType.DMA,
            pltpu.SemaphoreType.DMA,
        ],
    )
    def gather_kernel(tabT_hbm, idx_hbm, out_hbm, tab_v, idx_v,
                      out_va, out_vb, sem_i, sem_o0, sem_o1):
        w = lax.axis_index("subcore") * 2 + lax.axis_index("core")
        p = lax.rem(w, 16)
        half = w // 16
        NT_H = NTAB // 2
        t0 = (p * NT_H) // 16
        sem_o = (sem_o0, sem_o1)
        out_bufs = (out_va, out_vb)

        def start_idx(t):
            pltpu.async_copy(idx_hbm.at[t], idx_v, sem_i)

        def wait_idx():
            pltpu.make_async_copy(idx_hbm.at[0], idx_v, sem_i).wait()

        def wait_out(buf):
            pltpu.make_async_copy(
                out_bufs[buf], out_hbm.at[0, pl.ds(0, BC)], sem_o[buf]).wait()

        start_idx(half * NT_H + t0)

        @pl.loop(0, NT_H)
        def _(k):
            t = half * NT_H + lax.rem(t0 + k, NT_H)

            for ei in range(2):
                e = p * 2 + ei
                pltpu.sync_copy(tabT_hbm.at[t, e], tab_v)
                if ei == 0:
                    wait_idx()

                for c in range(NCHUNK):
                    cc = ei * NCHUNK + c
                    buf = cc % 2
                    if cc >= 2:
                        wait_out(buf)
                    else:
                        @pl.when(k > 0)
                        def _():
                            wait_out(buf)

                    ob = out_bufs[buf]

                    @plsc.parallel_loop(0, BC // 16, unroll=16)
                    def _(i):
                        iv = idx_v[pl.ds(c * BC + i * 16, 16)]
                        ob[pl.ds(i * 16, 16)] = plsc.load_gather(tab_v, [iv])

                    pltpu.async_copy(
                        ob, out_hbm.at[t * EDIM + e, pl.ds(c * BC, BC)],
                        sem_o[buf])

            @pl.when(k < NT_H - 1)
            def _():
                start_idx(half * NT_H + lax.rem(t0 + k + 1, NT_H))

        wait_out(0)
        wait_out(1)

    return gather_kernel(tabT, idx)


def _mlp_t_body(xT_ref, numT_ref, w1cT_ref, w1nT_ref, b1_ref, w2T_ref, b2_ref,
                w3T_ref, b3_ref, w4T_ref, b4_ref, outT_ref):
    h = jnp.dot(w1cT_ref[...], xT_ref[...], preferred_element_type=jnp.float32)
    h = h + jnp.dot(w1nT_ref[...], numT_ref[...], preferred_element_type=jnp.float32)
    h = jnp.maximum(h + b1_ref[...], 0.0)
    h = jnp.maximum(
        jnp.dot(w2T_ref[...], h, preferred_element_type=jnp.float32) + b2_ref[...], 0.0)
    h = jnp.maximum(
        jnp.dot(w3T_ref[...], h, preferred_element_type=jnp.float32) + b3_ref[...], 0.0)
    outT_ref[...] = jnp.dot(w4T_ref[...], h, preferred_element_type=jnp.float32) + b4_ref[...]


def _mlp_t(xT, numT, w1cT, w1nT, b1c, w2T, b2c, w3T, b3c, w4T, b4c):
    nblk = B // BN
    full = lambda shape: pl.BlockSpec(shape, lambda i: (0, 0))
    return pl.pallas_call(
        _mlp_t_body,
        grid=(nblk,),
        in_specs=[
            pl.BlockSpec((CAT_DIM, BN), lambda i: (0, i)),
            pl.BlockSpec((NUM_DENSE, BN), lambda i: (0, i)),
            full((512, CAT_DIM)),
            full((512, NUM_DENSE)),
            full((512, 1)),
            full((256, 512)),
            full((256, 1)),
            full((128, 256)),
            full((128, 1)),
            full((1, 128)),
            full((1, 1)),
        ],
        out_specs=pl.BlockSpec((1, BN), lambda i: (0, i)),
        out_shape=jax.ShapeDtypeStruct((1, B), jnp.float32),
    )(xT, numT, w1cT, w1nT, b1c, w2T, b2c, w3T, b3c, w4T, b4c)


def kernel(numerical_features, cat_features, tables, W1, b1, W2, b2, W3, b3, W4, b4):
    idx = jnp.mod(cat_features[:, :, 0], VOCAB)
    tabT = jnp.transpose(tables, (0, 2, 1))
    xT = _sc_gather_t(tabT, idx)
    numT = numerical_features.T
    outT = _mlp_t(
        xT,
        numT,
        W1[:CAT_DIM].T,
        W1[CAT_DIM:].T,
        b1.reshape(-1, 1),
        W2.T,
        b2.reshape(-1, 1),
        W3.T,
        b3.reshape(-1, 1),
        W4.T,
        b4.reshape(-1, 1),
    )
    return outT.reshape(B, 1)


# speedup vs baseline: 1.0196x; 1.0196x over previous
import functools

import jax
import jax.numpy as jnp
from jax import lax
from jax.experimental import pallas as pl
from jax.experimental.pallas import tpu as pltpu
from jax.experimental.pallas import tpu_sc as plsc

VOCAB = 100000
EDIM = 32
NTAB = 26
B = 16384
NUM_DENSE = 13
CAT_DIM = NTAB * EDIM

BC = 4096
NCHUNK = B // BC
BN = 2048


def _sc_gather_t(tabT, idx):
    mesh = plsc.VectorSubcoreMesh(core_axis_name="core", subcore_axis_name="subcore")

    @functools.partial(
        pl.kernel,
        out_type=jax.ShapeDtypeStruct((CAT_DIM, B), jnp.float32),
        mesh=mesh,
        compiler_params=pltpu.CompilerParams(needs_layout_passes=False),
        scratch_types=[
            pltpu.VMEM((VOCAB,), jnp.float32),
            pltpu.VMEM((B,), jnp.int32),
            pltpu.VMEM((BC,), jnp.float32),
            pltpu.VMEM((BC,), jnp.float32),
            pltpu.SemaphoreType.DMA,
            pltpu.SemaphoreType.DMA,
            pltpu.SemaphoreType.DMA,
        ],
    )
    def gather_kernel(tabT_hbm, idx_hbm, out_hbm, tab_v, idx_v,
                      out_va, out_vb, sem_i, sem_o0, sem_o1):
        w = lax.axis_index("subcore") * 2 + lax.axis_index("core")
        p = lax.rem(w, 16)
        half = w // 16
        NT_H = NTAB // 2
        t0 = (p * NT_H) // 16
        sem_o = (sem_o0, sem_o1)
        out_bufs = (out_va, out_vb)

        def start_idx(t):
            pltpu.async_copy(idx_hbm.at[t], idx_v, sem_i)

        def wait_idx():
            pltpu.make_async_copy(idx_hbm.at[0], idx_v, sem_i).wait()

        def wait_out(buf):
            pltpu.make_async_copy(
                out_bufs[buf], out_hbm.at[0, pl.ds(0, BC)], sem_o[buf]).wait()

        start_idx(half * NT_H + t0)

        @pl.loop(0, NT_H)
        def _(k):
            t = half * NT_H + lax.rem(t0 + k, NT_H)

            for ei in range(2):
                e = p * 2 + ei
                pltpu.sync_copy(tabT_hbm.at[t, e], tab_v)
                if ei == 0:
                    wait_idx()

                for c in range(NCHUNK):
                    cc = ei * NCHUNK + c
                    buf = cc % 2
                    if cc >= 2:
                        wait_out(buf)
                    else:
                        @pl.when(k > 0)
                        def _():
                            wait_out(buf)

                    ob = out_bufs[buf]

                    @plsc.parallel_loop(0, BC // 16, unroll=16)
                    def _(i):
                        iv = idx_v[pl.ds(c * BC + i * 16, 16)]
                        ob[pl.ds(i * 16, 16)] = plsc.load_gather(tab_v, [iv])

                    pltpu.async_copy(
                        ob, out_hbm.at[t * EDIM + e, pl.ds(c * BC, BC)],
                        sem_o[buf])

            @pl.when(k < NT_H - 1)
            def _():
                start_idx(half * NT_H + lax.rem(t0 + k + 1, NT_H))

        wait_out(0)
        wait_out(1)

    return gather_kernel(tabT, idx)


def _mlp_t_body(xT_ref, numT_ref, w1cT_ref, w1nT_ref, b1_ref, w2T_ref, b2_ref,
                w3T_ref, b3_ref, w4T_ref, b4_ref, outT_ref):
    h = jnp.dot(w1cT_ref[...], xT_ref[...], preferred_element_type=jnp.float32)
    h = h + jnp.dot(w1nT_ref[...], numT_ref[...], preferred_element_type=jnp.float32)
    h = jnp.maximum(h + b1_ref[...], 0.0)
    h = jnp.maximum(
        jnp.dot(w2T_ref[...], h, preferred_element_type=jnp.float32) + b2_ref[...], 0.0)
    h = jnp.maximum(
        jnp.dot(w3T_ref[...], h, preferred_element_type=jnp.float32) + b3_ref[...], 0.0)
    outT_ref[...] = jnp.dot(w4T_ref[...], h, preferred_element_type=jnp.float32) + b4_ref[...]


def _mlp_t(xT, numT, w1cT, w1nT, b1c, w2T, b2c, w3T, b3c, w4T, b4c):
    nblk = B // BN
    full = lambda shape: pl.BlockSpec(shape, lambda i: (0, 0))
    return pl.pallas_call(
        _mlp_t_body,
        grid=(nblk,),
        in_specs=[
            pl.BlockSpec((CAT_DIM, BN), lambda i: (0, i)),
            pl.BlockSpec((NUM_DENSE, BN), lambda i: (0, i)),
            full((512, CAT_DIM)),
            full((512, NUM_DENSE)),
            full((512, 1)),
            full((256, 512)),
            full((256, 1)),
            full((128, 256)),
            full((128, 1)),
            full((1, 128)),
            full((1, 1)),
        ],
        out_specs=pl.BlockSpec((1, BN), lambda i: (0, i)),
        out_shape=jax.ShapeDtypeStruct((1, B), jnp.float32),
    )(xT, numT, w1cT, w1nT, b1c, w2T, b2c, w3T, b3c, w4T, b4c)


def kernel(numerical_features, cat_features, tables, W1, b1, W2, b2, W3, b3, W4, b4):
    idx = jnp.mod(cat_features[:, :, 0], VOCAB)
    tabT = jnp.transpose(tables, (0, 2, 1))
    xT = _sc_gather_t(tabT, idx)
    numT = numerical_features.T
    outT = _mlp_t(
        xT,
        numT,
        W1[:CAT_DIM].T,
        W1[CAT_DIM:].T,
        b1.reshape(-1, 1),
        W2.T,
        b2.reshape(-1, 1),
        W3.T,
        b3.reshape(-1, 1),
        W4.T,
        b4.reshape(-1, 1),
    )
    return outT.reshape(B, 1)
